# initial kernel scaffold (unmeasured)
import jax
import jax.numpy as jnp
from jax import lax
from jax.experimental import pallas as pl
from jax.experimental.pallas import tpu as pltpu

N_Y = 4


def kernel(partial, resid, gamma):
    _, m, d = partial.shape
    mc = m // N_Y

    p = partial.reshape(N_Y, mc, d)
    r = resid.reshape(N_Y, mc, d)
    g = gamma.reshape(1, d)

    def body(p_ref, r_ref, g_ref, out_ref, rbuf, rs_send, rs_recv, ag_send, ag_recv):
        my_x = lax.axis_index("x")
        my_y = lax.axis_index("y")
        my_z = lax.axis_index("z")
        right = (my_y + 1) % N_Y
        left = (my_y + N_Y - 1) % N_Y

        barrier_sem = pltpu.get_barrier_semaphore()
        for nbr in (left, right):
            pl.semaphore_signal(
                barrier_sem,
                inc=1,
                device_id=(my_x, nbr, my_z),
                device_id_type=pl.DeviceIdType.MESH,
            )
        pl.semaphore_wait(barrier_sem, 2)

        for s in range(N_Y - 1):
            c = (my_y - s) % N_Y
            if s == 0:
                src = p_ref.at[c]
            else:
                rbuf[s - 1, :, :] = rbuf[s - 1, :, :] + p_ref[c, :, :]
                src = rbuf.at[s - 1]
            rdma = pltpu.make_async_remote_copy(
                src_ref=src,
                dst_ref=rbuf.at[s],
                send_sem=rs_send.at[s],
                recv_sem=rs_recv.at[s],
                device_id=(my_x, right, my_z),
                device_id_type=pl.DeviceIdType.MESH,
            )
            rdma.start()
            rdma.wait()

        o = (my_y + 1) % N_Y
        y = rbuf[N_Y - 2, :, :] + p_ref[o, :, :] + r_ref[o, :, :]
        rms = jnp.sqrt(jnp.mean(y * y, axis=-1, keepdims=True) + 1e-6)
        out_ref[o, :, :] = (y / rms) * g_ref[:, :]

        for s in range(N_Y - 1):
            send_idx = (my_y + 1 - s) % N_Y
            rdma = pltpu.make_async_remote_copy(
                src_ref=out_ref.at[send_idx],
                dst_ref=out_ref.at[send_idx],
                send_sem=ag_send.at[s],
                recv_sem=ag_recv.at[s],
                device_id=(my_x, right, my_z),
                device_id_type=pl.DeviceIdType.MESH,
            )
            rdma.start()
            rdma.wait()

    out = pl.pallas_call(
        body,
        out_shape=jax.ShapeDtypeStruct((N_Y, mc, d), jnp.float32),
        in_specs=[
            pl.BlockSpec(memory_space=pltpu.VMEM),
            pl.BlockSpec(memory_space=pltpu.VMEM),
            pl.BlockSpec(memory_space=pltpu.VMEM),
        ],
        out_specs=pl.BlockSpec(memory_space=pltpu.VMEM),
        scratch_shapes=[
            pltpu.VMEM((N_Y - 1, mc, d), jnp.float32),
            pltpu.SemaphoreType.DMA((N_Y - 1,)),
            pltpu.SemaphoreType.DMA((N_Y - 1,)),
            pltpu.SemaphoreType.DMA((N_Y - 1,)),
            pltpu.SemaphoreType.DMA((N_Y - 1,)),
        ],
        compiler_params=pltpu.CompilerParams(collective_id=0),
    )(p, r, g)

    return out.reshape(m, d)


# baseline (device time: 311463 ns/iter reference)
import jax
import jax.numpy as jnp
from jax import lax
from jax.experimental import pallas as pl
from jax.experimental.pallas import tpu as pltpu

N_Y = 4


def kernel(partial, resid, gamma):
    _, m, d = partial.shape
    mc = m // N_Y

    p = partial.reshape(N_Y, mc, d)
    r = resid.reshape(N_Y, mc, d)
    g = gamma.reshape(1, d)

    def body(
        p_ref,
        r_ref,
        g_ref,
        out_ref,
        rbuf,
        rchunk,
        rs_send,
        rs_recv,
        ag_send,
        ag_recv,
        rcopy_sem,
    ):
        my_x = lax.axis_index("x")
        my_y = lax.axis_index("y")
        my_z = lax.axis_index("z")
        right = (my_y + 1) % N_Y
        left = (my_y + N_Y - 1) % N_Y

        barrier_sem = pltpu.get_barrier_semaphore()
        for nbr in (left, right):
            pl.semaphore_signal(
                barrier_sem,
                inc=1,
                device_id=(my_x, nbr, my_z),
                device_id_type=pl.DeviceIdType.MESH,
            )
        pl.semaphore_wait(barrier_sem, 2)

        o = (my_y + 1) % N_Y
        rcopy = pltpu.make_async_copy(r_ref.at[o], rchunk, rcopy_sem)
        rcopy.start()

        for s in range(N_Y - 1):
            c = (my_y - s) % N_Y
            if s == 0:
                src = p_ref.at[c]
            else:
                rbuf[s - 1, :, :] = rbuf[s - 1, :, :] + p_ref[c, :, :]
                src = rbuf.at[s - 1]
            rdma = pltpu.make_async_remote_copy(
                src_ref=src,
                dst_ref=rbuf.at[s],
                send_sem=rs_send.at[s],
                recv_sem=rs_recv.at[s],
                device_id=(my_x, right, my_z),
                device_id_type=pl.DeviceIdType.MESH,
            )
            rdma.start()
            rdma.wait()

        rcopy.wait()
        y = rbuf[N_Y - 2, :, :] + p_ref[o, :, :] + rchunk[:, :]
        rms = jnp.sqrt(jnp.mean(y * y, axis=-1, keepdims=True) + 1e-6)
        out_ref[o, :, :] = (y / rms) * g_ref[:, :]

        for s in range(N_Y - 1):
            send_idx = (my_y + 1 - s) % N_Y
            rdma = pltpu.make_async_remote_copy(
                src_ref=out_ref.at[send_idx],
                dst_ref=out_ref.at[send_idx],
                send_sem=ag_send.at[s],
                recv_sem=ag_recv.at[s],
                device_id=(my_x, right, my_z),
                device_id_type=pl.DeviceIdType.MESH,
            )
            rdma.start()
            rdma.wait()

    out = pl.pallas_call(
        body,
        out_shape=jax.ShapeDtypeStruct((N_Y, mc, d), jnp.float32),
        in_specs=[
            pl.BlockSpec(memory_space=pltpu.VMEM),
            pl.BlockSpec(memory_space=pltpu.MemorySpace.HBM),
            pl.BlockSpec(memory_space=pltpu.VMEM),
        ],
        out_specs=pl.BlockSpec(memory_space=pltpu.VMEM),
        scratch_shapes=[
            pltpu.VMEM((N_Y - 1, mc, d), jnp.float32),
            pltpu.VMEM((mc, d), jnp.float32),
            pltpu.SemaphoreType.DMA((N_Y - 1,)),
            pltpu.SemaphoreType.DMA((N_Y - 1,)),
            pltpu.SemaphoreType.DMA((N_Y - 1,)),
            pltpu.SemaphoreType.DMA((N_Y - 1,)),
            pltpu.SemaphoreType.DMA,
        ],
        compiler_params=pltpu.CompilerParams(
            collective_id=0, vmem_limit_bytes=100 * 1024 * 1024
        ),
    )(p, r, g)

    return out.reshape(m, d)


# device time: 196485 ns/iter; 1.5852x vs baseline; 1.5852x over previous
import jax
import jax.numpy as jnp
from jax import lax
from jax.experimental import pallas as pl
from jax.experimental.pallas import tpu as pltpu

N_Y = 4
N_X = 2


def kernel(partial, resid, gamma):
    _, m, d = partial.shape
    mc = m // (N_X * N_Y)

    p = partial.reshape(N_X, N_Y, mc, d)
    r = resid.reshape(N_X, N_Y, mc, d)
    g = gamma.reshape(1, d)

    def body(
        p_ref,
        r_ref,
        g_ref,
        out_ref,
        pbuf,
        rbuf,
        rchunk,
        rs_send,
        rs_recv,
        ag_send,
        ag_recv,
        x_send,
        x_recv,
        copy_sems,
    ):
        my_x = lax.axis_index("x")
        my_y = lax.axis_index("y")
        my_z = lax.axis_index("z")
        right = (my_y + 1) % N_Y
        left = (my_y + N_Y - 1) % N_Y
        xpeer = 1 - my_x
        o = (my_y + 1) % N_Y

        barrier_sem = pltpu.get_barrier_semaphore()
        for nbr in ((my_x, left, my_z), (my_x, right, my_z), (xpeer, my_y, my_z)):
            pl.semaphore_signal(
                barrier_sem, inc=1, device_id=nbr,
                device_id_type=pl.DeviceIdType.MESH,
            )
        pl.semaphore_wait(barrier_sem, 3)

        pcopies = []
        for s in range(N_Y - 1):
            c = (my_y - 1 - s) % N_Y if s < N_Y - 2 else o
            cp = pltpu.make_async_copy(
                p_ref.at[my_x, c], pbuf.at[s], copy_sems.at[s]
            )
            cp.start()
            pcopies.append(cp)
        rcopy = pltpu.make_async_copy(
            r_ref.at[my_x, o], rchunk, copy_sems.at[N_Y - 1]
        )
        rcopy.start()

        for s in range(N_Y - 1):
            if s == 0:
                src = p_ref.at[my_x, my_y]
            else:
                pcopies[s - 1].wait()
                rbuf[s - 1, :, :] = rbuf[s - 1, :, :] + pbuf[s - 1, :, :]
                src = rbuf.at[s - 1]
            rdma = pltpu.make_async_remote_copy(
                src_ref=src,
                dst_ref=rbuf.at[s],
                send_sem=rs_send.at[s],
                recv_sem=rs_recv.at[s],
                device_id=(my_x, right, my_z),
                device_id_type=pl.DeviceIdType.MESH,
            )
            rdma.start()
            rdma.wait()

        pcopies[N_Y - 2].wait()
        rcopy.wait()
        y = rbuf[N_Y - 2, :, :] + pbuf[N_Y - 2, :, :] + rchunk[:, :]
        rms = jnp.sqrt(jnp.mean(y * y, axis=-1, keepdims=True) + 1e-6)
        out_ref[my_x, o, :, :] = (y / rms) * g_ref[:, :]

        def x_exchange(idx, k):
            rdma = pltpu.make_async_remote_copy(
                src_ref=out_ref.at[my_x, idx],
                dst_ref=out_ref.at[my_x, idx],
                send_sem=x_send.at[k],
                recv_sem=x_recv.at[k],
                device_id=(xpeer, my_y, my_z),
                device_id_type=pl.DeviceIdType.MESH,
            )
            rdma.start()
            return rdma

        xdmas = [x_exchange(o, 0)]

        for s in range(N_Y - 1):
            send_idx = (my_y + 1 - s) % N_Y
            rdma = pltpu.make_async_remote_copy(
                src_ref=out_ref.at[my_x, send_idx],
                dst_ref=out_ref.at[my_x, send_idx],
                send_sem=ag_send.at[s],
                recv_sem=ag_recv.at[s],
                device_id=(my_x, right, my_z),
                device_id_type=pl.DeviceIdType.MESH,
            )
            rdma.start()
            rdma.wait()
            xdmas.append(x_exchange((my_y - s) % N_Y, s + 1))

        for x_rdma in xdmas:
            x_rdma.wait()

    out = pl.pallas_call(
        body,
        out_shape=jax.ShapeDtypeStruct((N_X, N_Y, mc, d), jnp.float32),
        in_specs=[
            pl.BlockSpec(memory_space=pltpu.MemorySpace.HBM),
            pl.BlockSpec(memory_space=pltpu.MemorySpace.HBM),
            pl.BlockSpec(memory_space=pltpu.MemorySpace.VMEM),
        ],
        out_specs=pl.BlockSpec(memory_space=pltpu.MemorySpace.VMEM),
        scratch_shapes=[
            pltpu.VMEM((N_Y - 1, mc, d), jnp.float32),
            pltpu.VMEM((N_Y - 1, mc, d), jnp.float32),
            pltpu.VMEM((mc, d), jnp.float32),
            pltpu.SemaphoreType.DMA((N_Y - 1,)),
            pltpu.SemaphoreType.DMA((N_Y - 1,)),
            pltpu.SemaphoreType.DMA((N_Y - 1,)),
            pltpu.SemaphoreType.DMA((N_Y - 1,)),
            pltpu.SemaphoreType.DMA((N_Y,)),
            pltpu.SemaphoreType.DMA((N_Y,)),
            pltpu.SemaphoreType.DMA((N_Y,)),
        ],
        compiler_params=pltpu.CompilerParams(
            collective_id=0, vmem_limit_bytes=100 * 1024 * 1024
        ),
    )(p, r, g)

    return out.reshape(m, d)


# device time: 158522 ns/iter; 1.9648x vs baseline; 1.2395x over previous
import jax
import jax.numpy as jnp
from jax import lax
from jax.experimental import pallas as pl
from jax.experimental.pallas import tpu as pltpu

N_Y = 4
N_X = 2
N_ZG = 2


def kernel(partial, resid, gamma):
    _, m, d = partial.shape
    mc = m // (N_X * N_ZG * N_Y)

    p = partial.reshape(N_X, N_ZG, N_Y, mc, d)
    r = resid.reshape(N_X, N_ZG, N_Y, mc, d)
    g = gamma.reshape(1, d)

    def body(
        p_ref,
        r_ref,
        g_ref,
        out_ref,
        pbuf,
        rbuf,
        rchunk,
        rs_send,
        rs_recv,
        ag_send,
        ag_recv,
        x_send,
        x_recv,
        z_send,
        z_recv,
        f_send,
        f_recv,
        copy_sems,
    ):
        my_x = lax.axis_index("x")
        my_y = lax.axis_index("y")
        my_z = lax.axis_index("z")
        right = (my_y + 1) % N_Y
        left = (my_y + N_Y - 1) % N_Y
        xpeer = 1 - my_x
        zg = my_z % N_ZG
        zpartner = my_z + 1 - 2 * zg
        o = (my_y + 1) % N_Y

        barrier_sem = pltpu.get_barrier_semaphore()
        for nbr in (
            (my_x, left, my_z),
            (my_x, right, my_z),
            (xpeer, my_y, my_z),
            (my_x, my_y, zpartner),
        ):
            pl.semaphore_signal(
                barrier_sem, inc=1, device_id=nbr,
                device_id_type=pl.DeviceIdType.MESH,
            )
        pl.semaphore_wait(barrier_sem, 4)

        pcopies = []
        for k in range(N_Y - 1):
            c = (my_y - 1 - k) % N_Y
            cp = pltpu.make_async_copy(
                p_ref.at[my_x, zg, c], pbuf.at[k], copy_sems.at[k]
            )
            cp.start()
            pcopies.append(cp)
        rcopy = pltpu.make_async_copy(
            r_ref.at[my_x, zg, o], rchunk, copy_sems.at[N_Y - 1]
        )
        rcopy.start()

        for s in range(N_Y - 1):
            if s == 0:
                src = p_ref.at[my_x, zg, my_y]
            else:
                pcopies[s - 1].wait()
                rbuf[s - 1, :, :] = rbuf[s - 1, :, :] + pbuf[s - 1, :, :]
                src = rbuf.at[s - 1]
            rdma = pltpu.make_async_remote_copy(
                src_ref=src,
                dst_ref=rbuf.at[s],
                send_sem=rs_send.at[s],
                recv_sem=rs_recv.at[s],
                device_id=(my_x, right, my_z),
                device_id_type=pl.DeviceIdType.MESH,
            )
            rdma.start()
            rdma.wait()

        pcopies[N_Y - 2].wait()
        rcopy.wait()
        y = rbuf[N_Y - 2, :, :] + pbuf[N_Y - 2, :, :] + rchunk[:, :]
        rms = jnp.sqrt(jnp.mean(y * y, axis=-1, keepdims=True) + 1e-6)
        out_ref[my_x, zg, o, :, :] = (y / rms) * g_ref[:, :]

        def exchange(idx, k, target, send_sems, recv_sems, src_x):
            rdma = pltpu.make_async_remote_copy(
                src_ref=out_ref.at[src_x, zg, idx],
                dst_ref=out_ref.at[src_x, zg, idx],
                send_sem=send_sems.at[k],
                recv_sem=recv_sems.at[k],
                device_id=target,
                device_id_type=pl.DeviceIdType.MESH,
            )
            rdma.start()
            return rdma

        x_target = (xpeer, my_y, my_z)
        z_target = (my_x, my_y, zpartner)

        def issue_xz(idx, k):
            xd = exchange(idx, k, x_target, x_send, x_recv, my_x)
            zd = exchange(idx, k, z_target, z_send, z_recv, my_x)
            return xd, zd

        xdmas, zdmas, fdmas = [], [], []
        chunk_idx = [o]
        xd, zd = issue_xz(o, 0)
        xdmas.append(xd)
        zdmas.append(zd)

        for s in range(N_Y - 1):
            send_idx = (my_y + 1 - s) % N_Y
            rdma = pltpu.make_async_remote_copy(
                src_ref=out_ref.at[my_x, zg, send_idx],
                dst_ref=out_ref.at[my_x, zg, send_idx],
                send_sem=ag_send.at[s],
                recv_sem=ag_recv.at[s],
                device_id=(my_x, right, my_z),
                device_id_type=pl.DeviceIdType.MESH,
            )
            rdma.start()
            rdma.wait()

            got = (my_y - s) % N_Y
            chunk_idx.append(got)
            xd, zd = issue_xz(got, s + 1)
            xdmas.append(xd)
            zdmas.append(zd)

            xdmas[s].wait()
            fdmas.append(
                exchange(chunk_idx[s], s, z_target, f_send, f_recv, xpeer)
            )

        xdmas[N_Y - 1].wait()
        fdmas.append(
            exchange(chunk_idx[N_Y - 1], N_Y - 1, z_target, f_send, f_recv, xpeer)
        )

        for rd in zdmas:
            rd.wait()
        for rd in fdmas:
            rd.wait()

    out = pl.pallas_call(
        body,
        out_shape=jax.ShapeDtypeStruct((N_X, N_ZG, N_Y, mc, d), jnp.float32),
        in_specs=[
            pl.BlockSpec(memory_space=pltpu.MemorySpace.HBM),
            pl.BlockSpec(memory_space=pltpu.MemorySpace.HBM),
            pl.BlockSpec(memory_space=pltpu.MemorySpace.VMEM),
        ],
        out_specs=pl.BlockSpec(memory_space=pltpu.MemorySpace.VMEM),
        scratch_shapes=[
            pltpu.VMEM((N_Y - 1, mc, d), jnp.float32),
            pltpu.VMEM((N_Y - 1, mc, d), jnp.float32),
            pltpu.VMEM((mc, d), jnp.float32),
            pltpu.SemaphoreType.DMA((N_Y - 1,)),
            pltpu.SemaphoreType.DMA((N_Y - 1,)),
            pltpu.SemaphoreType.DMA((N_Y - 1,)),
            pltpu.SemaphoreType.DMA((N_Y - 1,)),
            pltpu.SemaphoreType.DMA((N_Y,)),
            pltpu.SemaphoreType.DMA((N_Y,)),
            pltpu.SemaphoreType.DMA((N_Y,)),
            pltpu.SemaphoreType.DMA((N_Y,)),
            pltpu.SemaphoreType.DMA((N_Y,)),
            pltpu.SemaphoreType.DMA((N_Y,)),
            pltpu.SemaphoreType.DMA((N_Y,)),
        ],
        compiler_params=pltpu.CompilerParams(
            collective_id=0, vmem_limit_bytes=100 * 1024 * 1024
        ),
    )(p, r, g)

    return out.reshape(m, d)


# device time: 136443 ns/iter; 2.2827x vs baseline; 1.1618x over previous
import jax
import jax.numpy as jnp
from jax import lax
from jax.experimental import pallas as pl
from jax.experimental.pallas import tpu as pltpu

N_Y = 4
N_X = 2
N_ZG = 2


def kernel(partial, resid, gamma):
    _, m, d = partial.shape
    mc = m // (N_X * N_ZG * N_Y)

    p = partial.reshape(N_X, N_ZG, N_Y, mc, d)
    r = resid.reshape(N_X, N_ZG, N_Y, mc, d)
    g = gamma.reshape(1, d)

    def body(
        p_ref,
        r_ref,
        g_ref,
        out_ref,
        pbuf,
        rbuf,
        rchunk,
        rs_send,
        rs_recv,
        ag_send,
        ag_recv,
        x_send,
        x_recv,
        z_send,
        z_recv,
        fz_send,
        fz_recv,
        fx_send,
        fx_recv,
        copy_sems,
    ):
        my_x = lax.axis_index("x")
        my_y = lax.axis_index("y")
        my_z = lax.axis_index("z")
        right = (my_y + 1) % N_Y
        left = (my_y + N_Y - 1) % N_Y
        xpeer = 1 - my_x
        zg = my_z % N_ZG
        zpartner = my_z + 1 - 2 * zg
        o = (my_y + 1) % N_Y

        barrier_sem = pltpu.get_barrier_semaphore()
        for nbr in (
            (my_x, left, my_z),
            (my_x, right, my_z),
            (xpeer, my_y, my_z),
            (my_x, my_y, zpartner),
        ):
            pl.semaphore_signal(
                barrier_sem, inc=1, device_id=nbr,
                device_id_type=pl.DeviceIdType.MESH,
            )
        pl.semaphore_wait(barrier_sem, 4)

        pcopies = []
        for k in range(N_Y - 1):
            c = (my_y - 1 - k) % N_Y
            cp = pltpu.make_async_copy(
                p_ref.at[my_x, zg, c], pbuf.at[k], copy_sems.at[k]
            )
            cp.start()
            pcopies.append(cp)
        rcopy = pltpu.make_async_copy(
            r_ref.at[my_x, zg, o], rchunk, copy_sems.at[N_Y - 1]
        )
        rcopy.start()

        for s in range(N_Y - 1):
            if s == 0:
                src = p_ref.at[my_x, zg, my_y]
            else:
                pcopies[s - 1].wait()
                rbuf[s - 1, :, :] = rbuf[s - 1, :, :] + pbuf[s - 1, :, :]
                src = rbuf.at[s - 1]
            rdma = pltpu.make_async_remote_copy(
                src_ref=src,
                dst_ref=rbuf.at[s],
                send_sem=rs_send.at[s],
                recv_sem=rs_recv.at[s],
                device_id=(my_x, right, my_z),
                device_id_type=pl.DeviceIdType.MESH,
            )
            rdma.start()
            rdma.wait()

        pcopies[N_Y - 2].wait()
        rcopy.wait()
        y = rbuf[N_Y - 2, :, :] + pbuf[N_Y - 2, :, :] + rchunk[:, :]
        rms = jnp.sqrt(jnp.mean(y * y, axis=-1, keepdims=True) + 1e-6)
        out_ref[my_x, zg, o, :, :] = (y / rms) * g_ref[:, :]

        def exchange(idx, k, target, send_sems, recv_sems, src_x, src_zg, rows=None):
            if rows is None:
                src = out_ref.at[src_x, src_zg, idx]
            else:
                src = out_ref.at[src_x, src_zg, idx, rows]
            rdma = pltpu.make_async_remote_copy(
                src_ref=src,
                dst_ref=src,
                send_sem=send_sems.at[k],
                recv_sem=recv_sems.at[k],
                device_id=target,
                device_id_type=pl.DeviceIdType.MESH,
            )
            rdma.start()
            return rdma

        x_target = (xpeer, my_y, my_z)
        z_target = (my_x, my_y, zpartner)

        def issue_xz(idx, k):
            xd = exchange(idx, k, x_target, x_send, x_recv, my_x, zg)
            zd = exchange(idx, k, z_target, z_send, z_recv, my_x, zg)
            return xd, zd

        half = mc // 2
        top = pl.ds(0, half)
        bot = pl.ds(half, half)

        def forward_xz(idx, k):
            return exchange(idx, k, z_target, fz_send, fz_recv, xpeer, zg, top)

        def forward_zx(idx, k):
            return exchange(idx, k, x_target, fx_send, fx_recv, my_x, 1 - zg, bot)

        xdmas, zdmas, fdmas = [], [], []
        chunk_idx = [o]
        xd, zd = issue_xz(o, 0)
        xdmas.append(xd)
        zdmas.append(zd)

        for s in range(N_Y - 1):
            send_idx = (my_y + 1 - s) % N_Y
            rdma = pltpu.make_async_remote_copy(
                src_ref=out_ref.at[my_x, zg, send_idx],
                dst_ref=out_ref.at[my_x, zg, send_idx],
                send_sem=ag_send.at[s],
                recv_sem=ag_recv.at[s],
                device_id=(my_x, right, my_z),
                device_id_type=pl.DeviceIdType.MESH,
            )
            rdma.start()
            rdma.wait()

            got = (my_y - s) % N_Y
            chunk_idx.append(got)
            xd, zd = issue_xz(got, s + 1)
            xdmas.append(xd)
            zdmas.append(zd)

            xdmas[s].wait()
            fdmas.append(forward_xz(chunk_idx[s], s))
            zdmas[s].wait()
            fdmas.append(forward_zx(chunk_idx[s], s))

        xdmas[N_Y - 1].wait()
        fdmas.append(forward_xz(chunk_idx[N_Y - 1], N_Y - 1))
        zdmas[N_Y - 1].wait()
        fdmas.append(forward_zx(chunk_idx[N_Y - 1], N_Y - 1))

        for rd in fdmas:
            rd.wait()

    out = pl.pallas_call(
        body,
        out_shape=jax.ShapeDtypeStruct((N_X, N_ZG, N_Y, mc, d), jnp.float32),
        in_specs=[
            pl.BlockSpec(memory_space=pltpu.MemorySpace.HBM),
            pl.BlockSpec(memory_space=pltpu.MemorySpace.HBM),
            pl.BlockSpec(memory_space=pltpu.MemorySpace.VMEM),
        ],
        out_specs=pl.BlockSpec(memory_space=pltpu.MemorySpace.VMEM),
        scratch_shapes=[
            pltpu.VMEM((N_Y - 1, mc, d), jnp.float32),
            pltpu.VMEM((N_Y - 1, mc, d), jnp.float32),
            pltpu.VMEM((mc, d), jnp.float32),
            pltpu.SemaphoreType.DMA((N_Y - 1,)),
            pltpu.SemaphoreType.DMA((N_Y - 1,)),
            pltpu.SemaphoreType.DMA((N_Y - 1,)),
            pltpu.SemaphoreType.DMA((N_Y - 1,)),
            pltpu.SemaphoreType.DMA((N_Y,)),
            pltpu.SemaphoreType.DMA((N_Y,)),
            pltpu.SemaphoreType.DMA((N_Y,)),
            pltpu.SemaphoreType.DMA((N_Y,)),
            pltpu.SemaphoreType.DMA((N_Y,)),
            pltpu.SemaphoreType.DMA((N_Y,)),
            pltpu.SemaphoreType.DMA((N_Y,)),
            pltpu.SemaphoreType.DMA((N_Y,)),
            pltpu.SemaphoreType.DMA((N_Y,)),
        ],
        compiler_params=pltpu.CompilerParams(
            collective_id=0, vmem_limit_bytes=100 * 1024 * 1024
        ),
    )(p, r, g)

    return out.reshape(m, d)


# device time: 132118 ns/iter; 2.3575x vs baseline; 1.0327x over previous
import jax
import jax.numpy as jnp
from jax import lax
from jax.experimental import pallas as pl
from jax.experimental.pallas import tpu as pltpu

N_Y = 4
N_X = 2
N_ZG = 2


def kernel(partial, resid, gamma):
    _, m, d = partial.shape
    mc = m // (N_X * N_ZG * N_Y)

    p = partial.reshape(N_X, N_ZG, N_Y, mc, d)
    r = resid.reshape(N_X, N_ZG, N_Y, mc, d)
    g = gamma.reshape(1, d)

    def body(
        p_ref,
        r_ref,
        g_ref,
        out_ref,
        pbuf,
        rbuf,
        rchunk,
        rs_send,
        rs_recv,
        ag_send,
        ag_recv,
        x_send,
        x_recv,
        z_send,
        z_recv,
        fz_send,
        fz_recv,
        fx_send,
        fx_recv,
        copy_sems,
    ):
        my_x = lax.axis_index("x")
        my_y = lax.axis_index("y")
        my_z = lax.axis_index("z")
        right = (my_y + 1) % N_Y
        left = (my_y + N_Y - 1) % N_Y
        xpeer = 1 - my_x
        zg = my_z % N_ZG
        zpartner = my_z + 1 - 2 * zg
        o = (my_y + 1) % N_Y

        barrier_sem = pltpu.get_barrier_semaphore()
        for nbr in (
            (my_x, left, my_z),
            (my_x, right, my_z),
            (xpeer, my_y, my_z),
            (my_x, my_y, zpartner),
        ):
            pl.semaphore_signal(
                barrier_sem, inc=1, device_id=nbr,
                device_id_type=pl.DeviceIdType.MESH,
            )
        pl.semaphore_wait(barrier_sem, 4)

        pcopies = []
        for k in range(N_Y - 1):
            c = (my_y - 1 - k) % N_Y
            cp = pltpu.make_async_copy(
                p_ref.at[my_x, zg, c], pbuf.at[k], copy_sems.at[k]
            )
            cp.start()
            pcopies.append(cp)
        rcopy = pltpu.make_async_copy(
            r_ref.at[my_x, zg, o], rchunk, copy_sems.at[N_Y - 1]
        )
        rcopy.start()

        half = mc // 2
        halves = (pl.ds(0, half), pl.ds(half, half))

        rs = {}
        for h, sl in enumerate(halves):
            rs[(0, h)] = pltpu.make_async_remote_copy(
                src_ref=p_ref.at[my_x, zg, my_y, sl],
                dst_ref=rbuf.at[0, sl],
                send_sem=rs_send.at[0, h],
                recv_sem=rs_recv.at[0, h],
                device_id=(my_x, right, my_z),
                device_id_type=pl.DeviceIdType.MESH,
            )
            rs[(0, h)].start()
        for s in range(1, N_Y - 1):
            pcopies[s - 1].wait()
            for h, sl in enumerate(halves):
                rs[(s - 1, h)].wait()
                rbuf[s - 1, sl, :] = rbuf[s - 1, sl, :] + pbuf[s - 1, sl, :]
                rs[(s, h)] = pltpu.make_async_remote_copy(
                    src_ref=rbuf.at[s - 1, sl],
                    dst_ref=rbuf.at[s, sl],
                    send_sem=rs_send.at[s, h],
                    recv_sem=rs_recv.at[s, h],
                    device_id=(my_x, right, my_z),
                    device_id_type=pl.DeviceIdType.MESH,
                )
                rs[(s, h)].start()
        for h in range(2):
            rs[(N_Y - 2, h)].wait()

        pcopies[N_Y - 2].wait()
        rcopy.wait()
        y = rbuf[N_Y - 2, :, :] + pbuf[N_Y - 2, :, :] + rchunk[:, :]
        rms = jnp.sqrt(jnp.mean(y * y, axis=-1, keepdims=True) + 1e-6)
        out_ref[my_x, zg, o, :, :] = (y / rms) * g_ref[:, :]

        def exchange(idx, k, target, send_sems, recv_sems, src_x, src_zg, rows=None):
            if rows is None:
                src = out_ref.at[src_x, src_zg, idx]
            else:
                src = out_ref.at[src_x, src_zg, idx, rows]
            rdma = pltpu.make_async_remote_copy(
                src_ref=src,
                dst_ref=src,
                send_sem=send_sems.at[k],
                recv_sem=recv_sems.at[k],
                device_id=target,
                device_id_type=pl.DeviceIdType.MESH,
            )
            rdma.start()
            return rdma

        x_target = (xpeer, my_y, my_z)
        z_target = (my_x, my_y, zpartner)

        def issue_xz(idx, k):
            xd = exchange(idx, k, x_target, x_send, x_recv, my_x, zg)
            zd = exchange(idx, k, z_target, z_send, z_recv, my_x, zg)
            return xd, zd

        def forward_xz(idx, k):
            return exchange(idx, k, z_target, fz_send, fz_recv, xpeer, zg, halves[0])

        def forward_zx(idx, k):
            return exchange(
                idx, k, x_target, fx_send, fx_recv, my_x, 1 - zg, halves[1]
            )

        def ag_rdma(idx, s, h, sl):
            rdma = pltpu.make_async_remote_copy(
                src_ref=out_ref.at[my_x, zg, idx, sl],
                dst_ref=out_ref.at[my_x, zg, idx, sl],
                send_sem=ag_send.at[s, h],
                recv_sem=ag_recv.at[s, h],
                device_id=(my_x, right, my_z),
                device_id_type=pl.DeviceIdType.MESH,
            )
            rdma.start()
            return rdma

        xdmas, zdmas, fdmas = [], [], []
        chunk_idx = [o]
        xd, zd = issue_xz(o, 0)
        xdmas.append(xd)
        zdmas.append(zd)

        agd = {}
        for h, sl in enumerate(halves):
            agd[(0, h)] = ag_rdma(o, 0, h, sl)

        for s in range(N_Y - 1):
            got = (my_y - s) % N_Y
            for h, sl in enumerate(halves):
                agd[(s, h)].wait()
                if s < N_Y - 2:
                    agd[(s + 1, h)] = ag_rdma(got, s + 1, h, sl)

            chunk_idx.append(got)
            xd, zd = issue_xz(got, s + 1)
            xdmas.append(xd)
            zdmas.append(zd)

            xdmas[s].wait()
            fdmas.append(forward_xz(chunk_idx[s], s))
            zdmas[s].wait()
            fdmas.append(forward_zx(chunk_idx[s], s))

        xdmas[N_Y - 1].wait()
        fdmas.append(forward_xz(chunk_idx[N_Y - 1], N_Y - 1))
        zdmas[N_Y - 1].wait()
        fdmas.append(forward_zx(chunk_idx[N_Y - 1], N_Y - 1))

        for rd in fdmas:
            rd.wait()

    out = pl.pallas_call(
        body,
        out_shape=jax.ShapeDtypeStruct((N_X, N_ZG, N_Y, mc, d), jnp.float32),
        in_specs=[
            pl.BlockSpec(memory_space=pltpu.MemorySpace.HBM),
            pl.BlockSpec(memory_space=pltpu.MemorySpace.HBM),
            pl.BlockSpec(memory_space=pltpu.MemorySpace.VMEM),
        ],
        out_specs=pl.BlockSpec(memory_space=pltpu.MemorySpace.VMEM),
        scratch_shapes=[
            pltpu.VMEM((N_Y - 1, mc, d), jnp.float32),
            pltpu.VMEM((N_Y - 1, mc, d), jnp.float32),
            pltpu.VMEM((mc, d), jnp.float32),
            pltpu.SemaphoreType.DMA((N_Y - 1, 2)),
            pltpu.SemaphoreType.DMA((N_Y - 1, 2)),
            pltpu.SemaphoreType.DMA((N_Y - 1, 2)),
            pltpu.SemaphoreType.DMA((N_Y - 1, 2)),
            pltpu.SemaphoreType.DMA((N_Y,)),
            pltpu.SemaphoreType.DMA((N_Y,)),
            pltpu.SemaphoreType.DMA((N_Y,)),
            pltpu.SemaphoreType.DMA((N_Y,)),
            pltpu.SemaphoreType.DMA((N_Y,)),
            pltpu.SemaphoreType.DMA((N_Y,)),
            pltpu.SemaphoreType.DMA((N_Y,)),
            pltpu.SemaphoreType.DMA((N_Y,)),
            pltpu.SemaphoreType.DMA((N_Y,)),
        ],
        compiler_params=pltpu.CompilerParams(
            collective_id=0, vmem_limit_bytes=100 * 1024 * 1024
        ),
    )(p, r, g)

    return out.reshape(m, d)


# device time: 125753 ns/iter; 2.4768x vs baseline; 1.0506x over previous
import jax
import jax.numpy as jnp
from jax import lax
from jax.experimental import pallas as pl
from jax.experimental.pallas import tpu as pltpu

N_Y = 4
N_X = 2
N_ZG = 2


def kernel(partial, resid, gamma):
    _, m, d = partial.shape
    mc = m // (N_X * N_ZG * N_Y)

    p = partial.reshape(N_X, N_ZG, N_Y, mc, d)
    r = resid.reshape(N_X, N_ZG, N_Y, mc, d)
    g = gamma.reshape(1, d)

    def body(
        p_ref,
        r_ref,
        g_ref,
        out_ref,
        pbuf,
        rbuf,
        rchunk,
        rs_send,
        rs_recv,
        ag_send,
        ag_recv,
        x_send,
        x_recv,
        z_send,
        z_recv,
        fz_send,
        fz_recv,
        fx_send,
        fx_recv,
        copy_sems,
    ):
        my_x = lax.axis_index("x")
        my_y = lax.axis_index("y")
        my_z = lax.axis_index("z")
        right = (my_y + 1) % N_Y
        left = (my_y + N_Y - 1) % N_Y
        xpeer = 1 - my_x
        zg = my_z % N_ZG
        zpartner = my_z + 1 - 2 * zg
        o = (my_y + 1) % N_Y

        barrier_sem = pltpu.get_barrier_semaphore()
        for nbr in (
            (my_x, left, my_z),
            (my_x, right, my_z),
            (xpeer, my_y, my_z),
            (my_x, my_y, zpartner),
        ):
            pl.semaphore_signal(
                barrier_sem, inc=1, device_id=nbr,
                device_id_type=pl.DeviceIdType.MESH,
            )
        pl.semaphore_wait(barrier_sem, 4)

        pcopies = []
        for k in range(N_Y - 1):
            c = (my_y - 1 - k) % N_Y
            cp = pltpu.make_async_copy(
                p_ref.at[my_x, zg, c], pbuf.at[k], copy_sems.at[k]
            )
            cp.start()
            pcopies.append(cp)
        rcopy = pltpu.make_async_copy(
            r_ref.at[my_x, zg, o], rchunk, copy_sems.at[N_Y - 1]
        )
        rcopy.start()

        half = mc // 2
        halves = (pl.ds(0, half), pl.ds(half, half))

        rs = {}
        for h, sl in enumerate(halves):
            rs[(0, h)] = pltpu.make_async_remote_copy(
                src_ref=p_ref.at[my_x, zg, my_y, sl],
                dst_ref=rbuf.at[0, sl],
                send_sem=rs_send.at[0, h],
                recv_sem=rs_recv.at[0, h],
                device_id=(my_x, right, my_z),
                device_id_type=pl.DeviceIdType.MESH,
            )
            rs[(0, h)].start()
        for s in range(1, N_Y - 1):
            pcopies[s - 1].wait()
            for h, sl in enumerate(halves):
                rs[(s - 1, h)].wait()
                rbuf[s - 1, sl, :] = rbuf[s - 1, sl, :] + pbuf[s - 1, sl, :]
                rs[(s, h)] = pltpu.make_async_remote_copy(
                    src_ref=rbuf.at[s - 1, sl],
                    dst_ref=rbuf.at[s, sl],
                    send_sem=rs_send.at[s, h],
                    recv_sem=rs_recv.at[s, h],
                    device_id=(my_x, right, my_z),
                    device_id_type=pl.DeviceIdType.MESH,
                )
                rs[(s, h)].start()
        def exchange(idx, k, h, target, send_sems, recv_sems, src_x, src_zg, sl):
            src = out_ref.at[src_x, src_zg, idx, sl]
            rdma = pltpu.make_async_remote_copy(
                src_ref=src,
                dst_ref=src,
                send_sem=send_sems.at[k, h],
                recv_sem=recv_sems.at[k, h],
                device_id=target,
                device_id_type=pl.DeviceIdType.MESH,
            )
            rdma.start()
            return rdma

        x_target = (xpeer, my_y, my_z)
        z_target = (my_x, my_y, zpartner)

        def issue_xz(idx, k, h, sl):
            xdmas[(k, h)] = exchange(idx, k, h, x_target, x_send, x_recv, my_x, zg, sl)
            zdmas[(k, h)] = exchange(idx, k, h, z_target, z_send, z_recv, my_x, zg, sl)

        def forward_xz(idx, k):
            return exchange(
                idx, k, 0, z_target, fz_send, fz_recv, xpeer, zg, halves[0]
            )

        def forward_zx(idx, k):
            return exchange(
                idx, k, 1, x_target, fx_send, fx_recv, my_x, 1 - zg, halves[1]
            )

        def ag_rdma(idx, s, h, sl):
            rdma = pltpu.make_async_remote_copy(
                src_ref=out_ref.at[my_x, zg, idx, sl],
                dst_ref=out_ref.at[my_x, zg, idx, sl],
                send_sem=ag_send.at[s, h],
                recv_sem=ag_recv.at[s, h],
                device_id=(my_x, right, my_z),
                device_id_type=pl.DeviceIdType.MESH,
            )
            rdma.start()
            return rdma

        xdmas, zdmas, fdmas = {}, {}, []
        agd = {}
        chunk_idx = [o]

        pcopies[N_Y - 2].wait()
        rcopy.wait()
        for h, sl in enumerate(halves):
            rs[(N_Y - 2, h)].wait()
            y = rbuf[N_Y - 2, sl, :] + pbuf[N_Y - 2, sl, :] + rchunk[sl, :]
            rms = jnp.sqrt(jnp.mean(y * y, axis=-1, keepdims=True) + 1e-6)
            out_ref[my_x, zg, o, sl, :] = (y / rms) * g_ref[:, :]
            agd[(0, h)] = ag_rdma(o, 0, h, sl)
            issue_xz(o, 0, h, sl)

        for s in range(N_Y - 1):
            got = (my_y - s) % N_Y
            for h, sl in enumerate(halves):
                agd[(s, h)].wait()
                if s < N_Y - 2:
                    agd[(s + 1, h)] = ag_rdma(got, s + 1, h, sl)
                issue_xz(got, s + 1, h, sl)

            chunk_idx.append(got)
            xdmas[(s, 0)].wait()
            fdmas.append(forward_xz(chunk_idx[s], s))
            zdmas[(s, 1)].wait()
            fdmas.append(forward_zx(chunk_idx[s], s))

        xdmas[(N_Y - 1, 0)].wait()
        fdmas.append(forward_xz(chunk_idx[N_Y - 1], N_Y - 1))
        zdmas[(N_Y - 1, 1)].wait()
        fdmas.append(forward_zx(chunk_idx[N_Y - 1], N_Y - 1))

        for k in range(N_Y):
            xdmas[(k, 1)].wait()
            zdmas[(k, 0)].wait()
        for rd in fdmas:
            rd.wait()

    out = pl.pallas_call(
        body,
        out_shape=jax.ShapeDtypeStruct((N_X, N_ZG, N_Y, mc, d), jnp.float32),
        in_specs=[
            pl.BlockSpec(memory_space=pltpu.MemorySpace.HBM),
            pl.BlockSpec(memory_space=pltpu.MemorySpace.HBM),
            pl.BlockSpec(memory_space=pltpu.MemorySpace.VMEM),
        ],
        out_specs=pl.BlockSpec(memory_space=pltpu.MemorySpace.VMEM),
        scratch_shapes=[
            pltpu.VMEM((N_Y - 1, mc, d), jnp.float32),
            pltpu.VMEM((N_Y - 1, mc, d), jnp.float32),
            pltpu.VMEM((mc, d), jnp.float32),
            pltpu.SemaphoreType.DMA((N_Y - 1, 2)),
            pltpu.SemaphoreType.DMA((N_Y - 1, 2)),
            pltpu.SemaphoreType.DMA((N_Y - 1, 2)),
            pltpu.SemaphoreType.DMA((N_Y - 1, 2)),
            pltpu.SemaphoreType.DMA((N_Y, 2)),
            pltpu.SemaphoreType.DMA((N_Y, 2)),
            pltpu.SemaphoreType.DMA((N_Y, 2)),
            pltpu.SemaphoreType.DMA((N_Y, 2)),
            pltpu.SemaphoreType.DMA((N_Y, 2)),
            pltpu.SemaphoreType.DMA((N_Y, 2)),
            pltpu.SemaphoreType.DMA((N_Y, 2)),
            pltpu.SemaphoreType.DMA((N_Y, 2)),
            pltpu.SemaphoreType.DMA((N_Y,)),
        ],
        compiler_params=pltpu.CompilerParams(
            collective_id=0, vmem_limit_bytes=100 * 1024 * 1024
        ),
    )(p, r, g)

    return out.reshape(m, d)
